# baseline (speedup 1.0000x reference)
import functools

import jax
import jax.numpy as jnp
from jax import lax
from jax.experimental import pallas as pl
from jax.experimental.pallas import tpu as pltpu
from jax.experimental.pallas import tpu_sc as plsc

B = 16384
EMB = 32
NC, NS = 2, 16
NW = NC * NS
BPW = B // NW
CHUNK = 128
NCHUNK = BPW // CHUNK


def _sc_gather(user_ids2d, movie_ids2d, gmf_u, gmf_m, mlp_u, mlp_m):
    mesh = plsc.VectorSubcoreMesh(core_axis_name="c", subcore_axis_name="s")
    out_type = tuple(jax.ShapeDtypeStruct((B, EMB), jnp.float32)
                     for _ in range(4))
    scratch = [
        pltpu.VMEM((NCHUNK, CHUNK), jnp.int32),
        pltpu.VMEM((NCHUNK, CHUNK), jnp.int32),
        pltpu.VMEM((BPW, EMB), jnp.float32),
        pltpu.VMEM((BPW, EMB), jnp.float32),
        pltpu.VMEM((BPW, EMB), jnp.float32),
        pltpu.VMEM((BPW, EMB), jnp.float32),
        pltpu.SemaphoreType.DMA,
    ]

    @functools.partial(
        pl.kernel, mesh=mesh, out_type=out_type, scratch_types=scratch,
        compiler_params=pltpu.CompilerParams(use_tc_tiling_on_sc=False))
    def k(uids_hbm, mids_hbm, gu_hbm, gm_hbm, mu_hbm, mm_hbm,
          ogu, ogm, omu, omm, idx_u, idx_m, bgu, bgm, bmu, bmm, sem):
        wid = lax.axis_index("s") * NC + lax.axis_index("c")
        irow = wid * NCHUNK
        pltpu.sync_copy(uids_hbm.at[pl.ds(irow, NCHUNK)], idx_u)
        pltpu.sync_copy(mids_hbm.at[pl.ds(irow, NCHUNK)], idx_m)
        copies = []
        for tbl, idx, buf in ((gu_hbm, idx_u, bgu), (gm_hbm, idx_m, bgm),
                              (mu_hbm, idx_u, bmu), (mm_hbm, idx_m, bmm)):
            for j in range(NCHUNK):
                copies.append(pltpu.async_copy(
                    tbl.at[idx.at[j]], buf.at[pl.ds(j * CHUNK, CHUNK)], sem))
        for cp in copies:
            cp.wait()
        base = wid * BPW
        for buf, out in ((bgu, ogu), (bgm, ogm), (bmu, omu), (bmm, omm)):
            pltpu.sync_copy(buf, out.at[pl.ds(base, BPW)])

    return k(user_ids2d, movie_ids2d, gmf_u, gmf_m, mlp_u, mlp_m)


BLK = 2048


def _tc_body(gu, gm, mu, mm, w1, b1, w2, b2, w3, b3, wo, bo, out):
    gmf = gu[...] * gm[...]
    x = jnp.concatenate([mu[...], mm[...]], axis=1)
    h = jnp.maximum(jnp.dot(x, w1[...], preferred_element_type=jnp.float32)
                    + b1[...], 0.0)
    h = jnp.maximum(jnp.dot(h, w2[...], preferred_element_type=jnp.float32)
                    + b2[...], 0.0)
    h = jnp.maximum(jnp.dot(h, w3[...], preferred_element_type=jnp.float32)
                    + b3[...], 0.0)
    comb = jnp.concatenate([gmf, h], axis=1)
    z = jnp.dot(comb, wo[...], preferred_element_type=jnp.float32) + bo[...]
    out[...] = jax.nn.sigmoid(z)


def _tc_dense(gu, gm, mu, mm, w1t, b1, w2t, b2, w3t, b3, wot, bo):
    row_spec = pl.BlockSpec((BLK, EMB), lambda i: (i, 0))

    def whole(shape):
        return pl.BlockSpec(shape, lambda i: tuple(0 for _ in shape))

    return pl.pallas_call(
        _tc_body,
        grid=(B // BLK,),
        in_specs=[row_spec, row_spec, row_spec, row_spec,
                  whole((64, 64)), whole((1, 64)),
                  whole((64, 32)), whole((1, 32)),
                  whole((32, 16)), whole((1, 16)),
                  whole((48, 1)), whole((1, 1))],
        out_specs=pl.BlockSpec((BLK, 1), lambda i: (i, 0)),
        out_shape=jax.ShapeDtypeStruct((B, 1), jnp.float32),
    )(gu, gm, mu, mm, w1t, b1, w2t, b2, w3t, b3, wot, bo)


def kernel(user_ids, movie_ids, gmf_user_emb, gmf_movie_emb,
           mlp_user_emb, mlp_movie_emb, W1, b1, W2, b2, W3, b3, Wo, bo):
    u2 = user_ids.reshape(B // CHUNK, CHUNK)
    m2 = movie_ids.reshape(B // CHUNK, CHUNK)
    gu, gm, mu, mm = _sc_gather(u2, m2, gmf_user_emb, gmf_movie_emb,
                                mlp_user_emb, mlp_movie_emb)
    out = _tc_dense(gu, gm, mu, mm,
                    W1.T, b1.reshape(1, 64),
                    W2.T, b2.reshape(1, 32),
                    W3.T, b3.reshape(1, 16),
                    Wo.T, bo.reshape(1, 1))
    return out.reshape(B)
